# initial kernel scaffold (unmeasured)
import jax
import jax.numpy as jnp
from jax import lax
from jax.experimental import pallas as pl
from jax.experimental.pallas import tpu as pltpu


def kernel(
    x,
):
    def body(*refs):
        pass

    out_shape = jax.ShapeDtypeStruct(..., jnp.float32)
    return pl.pallas_call(body, out_shape=out_shape)(...)


# baseline (device time: 19707 ns/iter reference)
import jax
import jax.numpy as jnp
from jax import lax
from jax.experimental import pallas as pl
from jax.experimental.pallas import tpu as pltpu


def kernel(x):
    _, m, nh = x.shape

    def body(x_ref, out_ref, send_buf, recv_buf, sum_buf, recv2_buf,
             send_sems, recv_sems):
        my_x = lax.axis_index("x")
        my_y = lax.axis_index("y")

        barrier_sem = pltpu.get_barrier_semaphore()
        pl.semaphore_signal(barrier_sem, inc=1,
                            device_id=(1 - my_x, my_y),
                            device_id_type=pl.DeviceIdType.MESH)
        pl.semaphore_signal(barrier_sem, inc=1,
                            device_id=(my_x, 1 - my_y),
                            device_id_type=pl.DeviceIdType.MESH)
        pl.semaphore_wait(barrier_sem, 2)

        send_buf[...] = x_ref[0].astype(jnp.bfloat16)

        rdma1 = pltpu.make_async_remote_copy(
            src_ref=send_buf,
            dst_ref=recv_buf,
            send_sem=send_sems.at[0],
            recv_sem=recv_sems.at[0],
            device_id=(1 - my_x, my_y),
            device_id_type=pl.DeviceIdType.MESH,
        )
        rdma1.start()
        rdma1.wait()

        sum_buf[...] = send_buf[...] + recv_buf[...]

        rdma2 = pltpu.make_async_remote_copy(
            src_ref=sum_buf,
            dst_ref=recv2_buf,
            send_sem=send_sems.at[1],
            recv_sem=recv_sems.at[1],
            device_id=(my_x, 1 - my_y),
            device_id_type=pl.DeviceIdType.MESH,
        )
        rdma2.start()
        rdma2.wait()

        @pl.when(my_y == 0)
        def _():
            out_ref[:, 0:nh] = sum_buf[...]
            out_ref[:, nh:2 * nh] = recv2_buf[...]

        @pl.when(my_y == 1)
        def _():
            out_ref[:, nh:2 * nh] = sum_buf[...]
            out_ref[:, 0:nh] = recv2_buf[...]

    return pl.pallas_call(
        body,
        out_shape=jax.ShapeDtypeStruct((m, 2 * nh), jnp.bfloat16),
        in_specs=[pl.BlockSpec(memory_space=pltpu.VMEM)],
        out_specs=pl.BlockSpec(memory_space=pltpu.VMEM),
        scratch_shapes=[
            pltpu.VMEM((m, nh), jnp.bfloat16),
            pltpu.VMEM((m, nh), jnp.bfloat16),
            pltpu.VMEM((m, nh), jnp.bfloat16),
            pltpu.VMEM((m, nh), jnp.bfloat16),
            pltpu.SemaphoreType.DMA((2,)),
            pltpu.SemaphoreType.DMA((2,)),
        ],
        compiler_params=pltpu.CompilerParams(collective_id=0),
    )(x)


# device time: 17784 ns/iter; 1.1081x vs baseline; 1.1081x over previous
import jax
import jax.numpy as jnp
from jax import lax
from jax.experimental import pallas as pl
from jax.experimental.pallas import tpu as pltpu


def kernel(x):
    _, m, nh = x.shape
    hm = m // 2

    def body(x_ref, out_ref, xb, rs_recv, q, send_sems, recv_sems):
        my_x = lax.axis_index("x")
        my_y = lax.axis_index("y")

        barrier_sem = pltpu.get_barrier_semaphore()
        for tgt in ((1 - my_x, my_y), (my_x, 1 - my_y), (1 - my_x, 1 - my_y)):
            pl.semaphore_signal(barrier_sem, inc=1, device_id=tgt,
                                device_id_type=pl.DeviceIdType.MESH)
        pl.semaphore_wait(barrier_sem, 3)

        xb[...] = x_ref[0].astype(jnp.bfloat16)

        rs = pltpu.make_async_remote_copy(
            src_ref=xb.at[pl.ds((1 - my_x) * hm, hm)],
            dst_ref=rs_recv,
            send_sem=send_sems.at[0],
            recv_sem=recv_sems.at[0],
            device_id=(1 - my_x, my_y),
            device_id_type=pl.DeviceIdType.MESH,
        )
        rs.start()
        rs.wait()

        q[...] = xb[pl.ds(my_x * hm, hm)] + rs_recv[...]
        out_ref[pl.ds(my_x * hm, hm), pl.ds(my_y * nh, nh)] = q[...]

        rdmas = []
        for idx, tgt in ((1, (1 - my_x, my_y)),
                         (2, (my_x, 1 - my_y)),
                         (3, (1 - my_x, 1 - my_y))):
            r = pltpu.make_async_remote_copy(
                src_ref=q,
                dst_ref=out_ref.at[pl.ds(my_x * hm, hm), pl.ds(my_y * nh, nh)],
                send_sem=send_sems.at[idx],
                recv_sem=recv_sems.at[idx],
                device_id=tgt,
                device_id_type=pl.DeviceIdType.MESH,
            )
            r.start()
            rdmas.append(r)
        for r in rdmas:
            r.wait()

    return pl.pallas_call(
        body,
        out_shape=jax.ShapeDtypeStruct((m, 2 * nh), jnp.bfloat16),
        in_specs=[pl.BlockSpec(memory_space=pltpu.VMEM)],
        out_specs=pl.BlockSpec(memory_space=pltpu.VMEM),
        scratch_shapes=[
            pltpu.VMEM((m, nh), jnp.bfloat16),
            pltpu.VMEM((hm, nh), jnp.bfloat16),
            pltpu.VMEM((hm, nh), jnp.bfloat16),
            pltpu.SemaphoreType.DMA((4,)),
            pltpu.SemaphoreType.DMA((4,)),
        ],
        compiler_params=pltpu.CompilerParams(collective_id=0),
    )(x)


# device time: 15676 ns/iter; 1.2571x vs baseline; 1.1345x over previous
import jax
import jax.numpy as jnp
from jax import lax
from jax.experimental import pallas as pl
from jax.experimental.pallas import tpu as pltpu


def kernel(x):
    _, m, nh = x.shape
    hm = m // 2

    def body(x_ref, out_ref, xb, rs_recv, yraw_recv, draw_recv, q,
             send_sems, recv_sems):
        my_x = lax.axis_index("x")
        my_y = lax.axis_index("y")
        x_tgt = (1 - my_x, my_y)
        y_tgt = (my_x, 1 - my_y)
        d_tgt = (1 - my_x, 1 - my_y)

        barrier_sem = pltpu.get_barrier_semaphore()
        for tgt in (x_tgt, y_tgt, d_tgt):
            pl.semaphore_signal(barrier_sem, inc=1, device_id=tgt,
                                device_id_type=pl.DeviceIdType.MESH)
        pl.semaphore_wait(barrier_sem, 3)

        xb[...] = x_ref[0].astype(jnp.bfloat16)

        r_x = pltpu.make_async_remote_copy(
            src_ref=xb.at[pl.ds((1 - my_x) * hm, hm)],
            dst_ref=rs_recv,
            send_sem=send_sems.at[0], recv_sem=recv_sems.at[0],
            device_id=x_tgt, device_id_type=pl.DeviceIdType.MESH,
        )
        r_y = pltpu.make_async_remote_copy(
            src_ref=xb.at[pl.ds((1 - my_x) * hm, hm)],
            dst_ref=yraw_recv,
            send_sem=send_sems.at[1], recv_sem=recv_sems.at[1],
            device_id=y_tgt, device_id_type=pl.DeviceIdType.MESH,
        )
        r_d = pltpu.make_async_remote_copy(
            src_ref=xb.at[pl.ds(my_x * hm, hm)],
            dst_ref=draw_recv,
            send_sem=send_sems.at[2], recv_sem=recv_sems.at[2],
            device_id=d_tgt, device_id_type=pl.DeviceIdType.MESH,
        )
        r_x.start()
        r_y.start()
        r_d.start()

        r_x.wait_recv()
        q[...] = xb[pl.ds(my_x * hm, hm)] + rs_recv[...]

        own_rows = pl.ds(my_x * hm, hm)
        own_cols = pl.ds(my_y * nh, nh)
        s_x = pltpu.make_async_remote_copy(
            src_ref=q, dst_ref=out_ref.at[own_rows, own_cols],
            send_sem=send_sems.at[3], recv_sem=recv_sems.at[3],
            device_id=x_tgt, device_id_type=pl.DeviceIdType.MESH,
        )
        s_y = pltpu.make_async_remote_copy(
            src_ref=q, dst_ref=out_ref.at[own_rows, own_cols],
            send_sem=send_sems.at[4], recv_sem=recv_sems.at[4],
            device_id=y_tgt, device_id_type=pl.DeviceIdType.MESH,
        )
        s_x.start()
        s_y.start()

        out_ref[own_rows, own_cols] = q[...]

        r_y.wait_recv()
        r_d.wait_recv()
        out_ref[pl.ds((1 - my_x) * hm, hm), pl.ds((1 - my_y) * nh, nh)] = (
            yraw_recv[...] + draw_recv[...]
        )

        s_x.wait()
        s_y.wait()
        r_x.wait_send()
        r_y.wait_send()
        r_d.wait_send()

    return pl.pallas_call(
        body,
        out_shape=jax.ShapeDtypeStruct((m, 2 * nh), jnp.bfloat16),
        in_specs=[pl.BlockSpec(memory_space=pltpu.VMEM)],
        out_specs=pl.BlockSpec(memory_space=pltpu.VMEM),
        scratch_shapes=[
            pltpu.VMEM((m, nh), jnp.bfloat16),
            pltpu.VMEM((hm, nh), jnp.bfloat16),
            pltpu.VMEM((hm, nh), jnp.bfloat16),
            pltpu.VMEM((hm, nh), jnp.bfloat16),
            pltpu.VMEM((hm, nh), jnp.bfloat16),
            pltpu.SemaphoreType.DMA((5,)),
            pltpu.SemaphoreType.DMA((5,)),
        ],
        compiler_params=pltpu.CompilerParams(collective_id=0),
    )(x)
